# baseline (speedup 1.0000x reference)
import functools

import jax
import jax.numpy as jnp
from jax import lax
from jax.experimental import pallas as pl
from jax.experimental.pallas import tpu as pltpu
from jax.experimental.pallas import tpu_sc as plsc

N = 10000
D = 128
E = 320000

NC = 2
NS = 16
NW = NC * NS

EPW = E // NW
EPP = 10240
NPAD = 10112
RPT = NPAD // NS
DR = NPAD // 128


def _count_degrees(deg_ref, dvec):
    s, _ = plsc.sort_key_val(dvec, dvec)
    pos = lax.iota(jnp.int32, 16)

    def take16(v, idx):
        return lax.gather(
            v, idx[:, None],
            lax.GatherDimensionNumbers(offset_dims=(), collapsed_slice_dims=(0,),
                                       start_index_map=(0,)),
            slice_sizes=(1,),
            mode=lax.GatherScatterMode.PROMISE_IN_BOUNDS)

    prv = take16(s, jnp.maximum(pos - 1, 0))
    nxt = take16(s, jnp.minimum(pos + 1, 15))
    is_start = (pos == 0) | (s != prv)
    is_last = (pos == 15) | (s != nxt)
    start = plsc.cummax(jnp.where(is_start, pos, 0))
    cnt = (pos - start + 1).astype(jnp.float32)
    plsc.addupdate_scatter(
        deg_ref,
        [lax.shift_right_logical(s, 7), lax.bitwise_and(s, 127)],
        cnt,
        mask=is_last,
    )


def _make_agg(with_deg, nph, c):
    nchunk = EPP // c
    pch = nchunk // nph
    mesh = plsc.VectorSubcoreMesh(core_axis_name="c", subcore_axis_name="s")
    out_type = [jax.ShapeDtypeStruct((NC, NPAD, 128), jnp.float32)]
    if with_deg:
        out_type.append(jax.ShapeDtypeStruct((NC, NS, DR, 128), jnp.float32))

    @functools.partial(
        pl.kernel,
        out_type=out_type,
        mesh=mesh,
        compiler_params=pltpu.CompilerParams(needs_layout_passes=False),
        scratch_types=[
            pltpu.VMEM((pch * c,), jnp.int32),
            pltpu.VMEM((pch, c), jnp.int32),
            pltpu.VMEM((2, c, 128), jnp.float32),
            pltpu.VMEM_SHARED((NPAD, 128), jnp.float32),
            pltpu.SemaphoreType.DMA,
            pltpu.SemaphoreType.DMA,
            pltpu.SemaphoreType.DMA,
            pltpu.SemaphoreType.DMA,
        ] + ([pltpu.VMEM((DR, 128), jnp.float32)] if with_deg else []),
    )
    def agg(table, srcs, dsts, zrows, *refs):
        if with_deg:
            out, out_deg, src_v, dst_v, rows_v, acc, g0, g1, s0, s1, deg_v = refs
        else:
            out, src_v, dst_v, rows_v, acc, g0, g1, s0, s1 = refs
            deg_v = None
        sems = (g0, g1)
        ssems = (s0, s1)
        cid = lax.axis_index("c")
        sid = lax.axis_index("s")
        wid = cid * NS + sid

        pltpu.sync_copy(zrows, acc.at[pl.ds(sid * RPT, RPT)])
        if with_deg:
            zero16 = jnp.zeros((16,), jnp.float32)

            def zbody(i, carry):
                for k in range(8):
                    deg_v[i, pl.ds(k * 16, 16)] = zero16
                return carry

            lax.fori_loop(0, DR, zbody, 0)
        plsc.subcore_barrier()

        def gather(j, b):
            return pltpu.async_copy(
                table.at[src_v.at[pl.ds(j * c, c)]], rows_v.at[b], sems[b]
            )

        def step(j, b):
            pltpu.make_async_copy(
                table.at[src_v.at[pl.ds(j * c, c)]], rows_v.at[b], sems[b]
            ).wait()
            pltpu.async_copy(rows_v.at[b], acc.at[dst_v.at[j]], ssems[b],
                             add=True)
            if with_deg:
                for k in range(c // 16):
                    _count_degrees(deg_v, dst_v[j, pl.ds(k * 16, 16)])
            pltpu.make_async_copy(
                rows_v.at[b], acc.at[dst_v.at[j]], ssems[b]
            ).wait()

        for p in range(nph):
            pltpu.sync_copy(srcs.at[wid, p], src_v)
            pltpu.sync_copy(dsts.at[wid, p], dst_v)
            gather(0, 0)
            gather(1, 1)

            def body(jj, carry):
                for b in range(2):
                    j = 2 * jj + b
                    step(j, b)

                    @pl.when(j + 2 < pch)
                    def _issue():
                        gather(j + 2, b)

                return carry

            lax.fori_loop(0, pch // 2, body, 0)
        plsc.subcore_barrier()

        pltpu.sync_copy(
            acc.at[pl.ds(sid * RPT, RPT)], out.at[cid, pl.ds(sid * RPT, RPT)]
        )
        if with_deg:
            pltpu.sync_copy(deg_v, out_deg.at[cid, sid])

    return agg


AGG1_NPH, AGG1_C = 2, 80
AGG2_NPH, AGG2_C = 2, 128
_agg_deg = _make_agg(True, AGG1_NPH, AGG1_C)
_agg_plain = _make_agg(False, AGG2_NPH, AGG2_C)


def _deg_to_col(degs):
    d80 = jnp.sum(degs, axis=(0, 1))
    pick = (lax.broadcasted_iota(jnp.int32, (NPAD, DR), 0) // 128
            == lax.broadcasted_iota(jnp.int32, (NPAD, DR), 1)).astype(jnp.float32)
    rows = lax.dot_general(pick, d80, (((1,), (0,)), ((), ())),
                           preferred_element_type=jnp.float32)
    lane = (lax.broadcasted_iota(jnp.int32, (NPAD, 128), 1)
            == lax.broadcasted_iota(jnp.int32, (NPAD, 128), 0) % 128)
    return jnp.sum(jnp.where(lane, rows, 0.0), axis=1, keepdims=True)


def _mm_t(a, w):
    return lax.dot_general(a, w, (((1,), (1,)), ((), ())),
                           preferred_element_type=jnp.float32)


def _root_body(a_ref, w_ref, b_ref, out_ref):
    out_ref[...] = _mm_t(a_ref[...], w_ref[...]) + b_ref[...][None, :]


def _root(a, w, b):
    return pl.pallas_call(
        _root_body,
        out_shape=jax.ShapeDtypeStruct((NPAD, 128), jnp.float32),
    )(a, w, b)


def _dense1_body(acc_ref, degs_ref, xr_ref, wl_ref, h_ref):
    s = acc_ref[0] + acc_ref[1]
    inv = 1.0 / jnp.maximum(_deg_to_col(degs_ref[...]), 1.0)
    mean = s * inv
    h = _mm_t(mean, wl_ref[...]) + xr_ref[...]
    h_ref[...] = jnp.maximum(h, 0.0)


def _dense2_body(acc_ref, degs_ref, hr_ref, wl_ref, wlin_ref, blin_ref,
                 out_ref):
    inv = 1.0 / jnp.maximum(_deg_to_col(degs_ref[...]), 1.0)
    mean = (acc_ref[0] + acc_ref[1]) * inv
    h2 = jnp.maximum(_mm_t(mean, wl_ref[...]) + hr_ref[...], 0.0)
    out_ref[...] = (_mm_t(h2, wlin_ref[...]) + blin_ref[...][None, :])[:N]


def _dense1(acc, degs, xr, Wl1):
    return pl.pallas_call(
        _dense1_body,
        out_shape=jax.ShapeDtypeStruct((NPAD, 128), jnp.float32),
    )(acc, degs, xr, Wl1)


def _dense2(acc, degs, hr, Wl2, Wlin, blin):
    return pl.pallas_call(
        _dense2_body,
        out_shape=jax.ShapeDtypeStruct((N, 128), jnp.float32),
    )(acc, degs, hr, Wl2, Wlin, blin)


def kernel(x, edge_index, Wl1, bl1, Wr1, Wl2, bl2, Wr2, Wlin, blin):
    src = edge_index[0].astype(jnp.int32)
    dst = edge_index[1].astype(jnp.int32)
    npad_e = EPP - EPW
    pad_iota = (jnp.arange(npad_e, dtype=jnp.int32)[None, :]
                + 37 * jnp.arange(NW, dtype=jnp.int32)[:, None])
    src_pad = (pad_iota * 41) % N
    dst_pad = N + pad_iota % (NPAD - N)
    srcs = jnp.concatenate([src.reshape(NW, EPW), src_pad], axis=1)
    dsts = jnp.concatenate([dst.reshape(NW, EPW), dst_pad], axis=1)

    xpad = jnp.concatenate([x, jnp.zeros((NPAD - N, 128), jnp.float32)], axis=0)
    zrows = jnp.zeros((RPT, 128), jnp.float32)

    acc1, degs = _agg_deg(
        xpad, srcs.reshape(NW, AGG1_NPH, EPP // AGG1_NPH),
        dsts.reshape(NW, AGG1_NPH, EPP // (AGG1_NPH * AGG1_C), AGG1_C), zrows)
    xr1 = _root(xpad, Wr1, bl1)
    h1 = _dense1(acc1, degs, xr1, Wl1)
    (acc2,) = _agg_plain(
        h1, srcs.reshape(NW, AGG2_NPH, EPP // AGG2_NPH),
        dsts.reshape(NW, AGG2_NPH, EPP // (AGG2_NPH * AGG2_C), AGG2_C), zrows)
    hr2 = _root(h1, Wr2, bl2)
    return _dense2(acc2, degs, hr2, Wl2, Wlin, blin)


# speedup vs baseline: 1.0214x; 1.0214x over previous
import functools

import jax
import jax.numpy as jnp
from jax import lax
from jax.experimental import pallas as pl
from jax.experimental.pallas import tpu as pltpu
from jax.experimental.pallas import tpu_sc as plsc

N = 10000
D = 128
E = 320000

NC = 2
NS = 16
NW = NC * NS

EPW = E // NW
EPP = 10240
NPAD = 10112
RPT = NPAD // NS
DR = NPAD // 128


def _count_degrees(deg_ref, dvec):
    s, _ = plsc.sort_key_val(dvec, dvec)
    pos = lax.iota(jnp.int32, 16)

    def take16(v, idx):
        return lax.gather(
            v, idx[:, None],
            lax.GatherDimensionNumbers(offset_dims=(), collapsed_slice_dims=(0,),
                                       start_index_map=(0,)),
            slice_sizes=(1,),
            mode=lax.GatherScatterMode.PROMISE_IN_BOUNDS)

    prv = take16(s, jnp.maximum(pos - 1, 0))
    nxt = take16(s, jnp.minimum(pos + 1, 15))
    is_start = (pos == 0) | (s != prv)
    is_last = (pos == 15) | (s != nxt)
    start = plsc.cummax(jnp.where(is_start, pos, 0))
    cnt = (pos - start + 1).astype(jnp.float32)
    plsc.addupdate_scatter(
        deg_ref,
        [lax.shift_right_logical(s, 7), lax.bitwise_and(s, 127)],
        cnt,
        mask=is_last,
    )


def _make_agg(with_deg, nph, c):
    nchunk = EPP // c
    pch = nchunk // nph
    mesh = plsc.VectorSubcoreMesh(core_axis_name="c", subcore_axis_name="s")
    out_type = [jax.ShapeDtypeStruct((NC, NPAD, 128), jnp.float32)]
    if with_deg:
        out_type.append(jax.ShapeDtypeStruct((NC, NS, DR, 128), jnp.float32))

    @functools.partial(
        pl.kernel,
        out_type=out_type,
        mesh=mesh,
        compiler_params=pltpu.CompilerParams(needs_layout_passes=False),
        scratch_types=[
            pltpu.VMEM((pch * c,), jnp.int32),
            pltpu.VMEM((pch, c), jnp.int32),
            pltpu.VMEM((2, c, 128), jnp.float32),
            pltpu.VMEM_SHARED((NPAD, 128), jnp.float32),
            pltpu.SemaphoreType.DMA,
            pltpu.SemaphoreType.DMA,
            pltpu.SemaphoreType.DMA,
            pltpu.SemaphoreType.DMA,
        ] + ([pltpu.VMEM((DR, 128), jnp.float32)] if with_deg else []),
    )
    def agg(table, srcs, dsts, zrows, *refs):
        if with_deg:
            out, out_deg, src_v, dst_v, rows_v, acc, g0, g1, s0, s1, deg_v = refs
        else:
            out, src_v, dst_v, rows_v, acc, g0, g1, s0, s1 = refs
            deg_v = None
        sems = (g0, g1)
        ssems = (s0, s1)
        cid = lax.axis_index("c")
        sid = lax.axis_index("s")
        wid = cid * NS + sid

        pltpu.sync_copy(zrows, acc.at[pl.ds(sid * RPT, RPT)])
        if with_deg:
            zero16 = jnp.zeros((16,), jnp.float32)

            def zbody(i, carry):
                for k in range(8):
                    deg_v[i, pl.ds(k * 16, 16)] = zero16
                return carry

            lax.fori_loop(0, DR, zbody, 0)
        plsc.subcore_barrier()

        def gather(j, b):
            return pltpu.async_copy(
                table.at[src_v.at[pl.ds(j * c, c)]], rows_v.at[b], sems[b]
            )

        def step(j, b):
            pltpu.make_async_copy(
                table.at[src_v.at[pl.ds(j * c, c)]], rows_v.at[b], sems[b]
            ).wait()
            pltpu.async_copy(rows_v.at[b], acc.at[dst_v.at[j]], ssems[b],
                             add=True)
            if with_deg:
                for k in range(c // 16):
                    _count_degrees(deg_v, dst_v[j, pl.ds(k * 16, 16)])
            pltpu.make_async_copy(
                rows_v.at[b], acc.at[dst_v.at[j]], ssems[b]
            ).wait()

        for p in range(nph):
            pltpu.sync_copy(srcs.at[wid, p], src_v)
            pltpu.sync_copy(dsts.at[wid, p], dst_v)
            gather(0, 0)
            gather(1, 1)

            def body(jj, carry):
                for b in range(2):
                    j = 2 * jj + b
                    step(j, b)

                    @pl.when(j + 2 < pch)
                    def _issue():
                        gather(j + 2, b)

                return carry

            lax.fori_loop(0, pch // 2, body, 0)
        plsc.subcore_barrier()

        pltpu.sync_copy(
            acc.at[pl.ds(sid * RPT, RPT)], out.at[cid, pl.ds(sid * RPT, RPT)]
        )
        if with_deg:
            pltpu.sync_copy(deg_v, out_deg.at[cid, sid])

    return agg


AGG1_NPH, AGG1_C = 4, 128
AGG2_NPH, AGG2_C = 2, 128
_agg_deg = _make_agg(True, AGG1_NPH, AGG1_C)
_agg_plain = _make_agg(False, AGG2_NPH, AGG2_C)


def _deg_to_col(degs):
    d80 = jnp.sum(degs, axis=(0, 1))
    pick = (lax.broadcasted_iota(jnp.int32, (NPAD, DR), 0) // 128
            == lax.broadcasted_iota(jnp.int32, (NPAD, DR), 1)).astype(jnp.float32)
    rows = lax.dot_general(pick, d80, (((1,), (0,)), ((), ())),
                           preferred_element_type=jnp.float32)
    lane = (lax.broadcasted_iota(jnp.int32, (NPAD, 128), 1)
            == lax.broadcasted_iota(jnp.int32, (NPAD, 128), 0) % 128)
    return jnp.sum(jnp.where(lane, rows, 0.0), axis=1, keepdims=True)


def _mm_t(a, w):
    return lax.dot_general(a, w, (((1,), (1,)), ((), ())),
                           preferred_element_type=jnp.float32)


def _root_body(a_ref, w_ref, b_ref, out_ref):
    out_ref[...] = _mm_t(a_ref[...], w_ref[...]) + b_ref[...][None, :]


def _root(a, w, b):
    return pl.pallas_call(
        _root_body,
        out_shape=jax.ShapeDtypeStruct((NPAD, 128), jnp.float32),
    )(a, w, b)


def _dense1_body(acc_ref, degs_ref, xr_ref, wl_ref, h_ref):
    s = acc_ref[0] + acc_ref[1]
    inv = 1.0 / jnp.maximum(_deg_to_col(degs_ref[...]), 1.0)
    mean = s * inv
    h = _mm_t(mean, wl_ref[...]) + xr_ref[...]
    h_ref[...] = jnp.maximum(h, 0.0)


def _dense2_body(acc_ref, degs_ref, hr_ref, wl_ref, wlin_ref, blin_ref,
                 out_ref):
    inv = 1.0 / jnp.maximum(_deg_to_col(degs_ref[...]), 1.0)
    mean = (acc_ref[0] + acc_ref[1]) * inv
    h2 = jnp.maximum(_mm_t(mean, wl_ref[...]) + hr_ref[...], 0.0)
    out_ref[...] = (_mm_t(h2, wlin_ref[...]) + blin_ref[...][None, :])[:N]


def _dense1(acc, degs, xr, Wl1):
    return pl.pallas_call(
        _dense1_body,
        out_shape=jax.ShapeDtypeStruct((NPAD, 128), jnp.float32),
    )(acc, degs, xr, Wl1)


def _dense2(acc, degs, hr, Wl2, Wlin, blin):
    return pl.pallas_call(
        _dense2_body,
        out_shape=jax.ShapeDtypeStruct((N, 128), jnp.float32),
    )(acc, degs, hr, Wl2, Wlin, blin)


def kernel(x, edge_index, Wl1, bl1, Wr1, Wl2, bl2, Wr2, Wlin, blin):
    src = edge_index[0].astype(jnp.int32)
    dst = edge_index[1].astype(jnp.int32)
    npad_e = EPP - EPW
    pad_iota = (jnp.arange(npad_e, dtype=jnp.int32)[None, :]
                + 37 * jnp.arange(NW, dtype=jnp.int32)[:, None])
    src_pad = (pad_iota * 41) % N
    dst_pad = N + pad_iota % (NPAD - N)
    srcs = jnp.concatenate([src.reshape(NW, EPW), src_pad], axis=1)
    dsts = jnp.concatenate([dst.reshape(NW, EPW), dst_pad], axis=1)

    xpad = jnp.concatenate([x, jnp.zeros((NPAD - N, 128), jnp.float32)], axis=0)
    zrows = jnp.zeros((RPT, 128), jnp.float32)

    acc1, degs = _agg_deg(
        xpad, srcs.reshape(NW, AGG1_NPH, EPP // AGG1_NPH),
        dsts.reshape(NW, AGG1_NPH, EPP // (AGG1_NPH * AGG1_C), AGG1_C), zrows)
    xr1 = _root(xpad, Wr1, bl1)
    h1 = _dense1(acc1, degs, xr1, Wl1)
    (acc2,) = _agg_plain(
        h1, srcs.reshape(NW, AGG2_NPH, EPP // AGG2_NPH),
        dsts.reshape(NW, AGG2_NPH, EPP // (AGG2_NPH * AGG2_C), AGG2_C), zrows)
    hr2 = _root(h1, Wr2, bl2)
    return _dense2(acc2, degs, hr2, Wl2, Wlin, blin)


# speedup vs baseline: 1.0311x; 1.0095x over previous
import functools

import jax
import jax.numpy as jnp
from jax import lax
from jax.experimental import pallas as pl
from jax.experimental.pallas import tpu as pltpu
from jax.experimental.pallas import tpu_sc as plsc

N = 10000
D = 128
E = 320000

NC = 2
NS = 16
NW = NC * NS

EPW = E // NW
EPP = 10240
NPAD = 10112
RPT = NPAD // NS
DR = NPAD // 128


def _count_degrees(deg_ref, dvec):
    s, _ = plsc.sort_key_val(dvec, dvec)
    pos = lax.iota(jnp.int32, 16)

    def take16(v, idx):
        return lax.gather(
            v, idx[:, None],
            lax.GatherDimensionNumbers(offset_dims=(), collapsed_slice_dims=(0,),
                                       start_index_map=(0,)),
            slice_sizes=(1,),
            mode=lax.GatherScatterMode.PROMISE_IN_BOUNDS)

    prv = take16(s, jnp.maximum(pos - 1, 0))
    nxt = take16(s, jnp.minimum(pos + 1, 15))
    is_start = (pos == 0) | (s != prv)
    is_last = (pos == 15) | (s != nxt)
    start = plsc.cummax(jnp.where(is_start, pos, 0))
    cnt = (pos - start + 1).astype(jnp.float32)
    plsc.addupdate_scatter(
        deg_ref,
        [lax.shift_right_logical(s, 7), lax.bitwise_and(s, 127)],
        cnt,
        mask=is_last,
    )


def _make_agg(with_deg, nph, c):
    nchunk = EPP // c
    pch = nchunk // nph
    mesh = plsc.VectorSubcoreMesh(core_axis_name="c", subcore_axis_name="s")
    out_type = [jax.ShapeDtypeStruct((NC, NPAD, 128), jnp.float32)]
    if with_deg:
        out_type.append(jax.ShapeDtypeStruct((NC, NS, DR, 128), jnp.float32))

    @functools.partial(
        pl.kernel,
        out_type=out_type,
        mesh=mesh,
        compiler_params=pltpu.CompilerParams(needs_layout_passes=False),
        scratch_types=[
            pltpu.VMEM((pch * c,), jnp.int32),
            pltpu.VMEM((pch, c), jnp.int32),
            pltpu.VMEM((2, c, 128), jnp.float32),
            pltpu.VMEM_SHARED((NPAD, 128), jnp.float32),
            pltpu.SemaphoreType.DMA,
            pltpu.SemaphoreType.DMA,
            pltpu.SemaphoreType.DMA,
            pltpu.SemaphoreType.DMA,
        ] + ([pltpu.VMEM((DR, 128), jnp.float32)] if with_deg else []),
    )
    def agg(table, srcs, dsts, zrows, *refs):
        if with_deg:
            out, out_deg, src_v, dst_v, rows_v, acc, g0, g1, s0, s1, deg_v = refs
        else:
            out, src_v, dst_v, rows_v, acc, g0, g1, s0, s1 = refs
            deg_v = None
        sems = (g0, g1)
        ssems = (s0, s1)
        cid = lax.axis_index("c")
        sid = lax.axis_index("s")
        wid = cid * NS + sid

        zc = pltpu.async_copy(zrows, acc.at[pl.ds(sid * RPT, RPT)], s0)
        i0 = pltpu.async_copy(srcs.at[wid, 0], src_v, g0)
        i1 = pltpu.async_copy(dsts.at[wid, 0], dst_v, g1)
        if with_deg:
            zero16 = jnp.zeros((16,), jnp.float32)

            def zbody(i, carry):
                for k in range(8):
                    deg_v[i, pl.ds(k * 16, 16)] = zero16
                return carry

            lax.fori_loop(0, DR, zbody, 0)
        zc.wait()
        i0.wait()
        i1.wait()
        plsc.subcore_barrier()

        def gather(j, b):
            return pltpu.async_copy(
                table.at[src_v.at[pl.ds(j * c, c)]], rows_v.at[b], sems[b]
            )

        def step(j, b):
            pltpu.make_async_copy(
                table.at[src_v.at[pl.ds(j * c, c)]], rows_v.at[b], sems[b]
            ).wait()
            pltpu.async_copy(rows_v.at[b], acc.at[dst_v.at[j]], ssems[b],
                             add=True)
            if with_deg:
                for k in range(c // 16):
                    _count_degrees(deg_v, dst_v[j, pl.ds(k * 16, 16)])
            pltpu.make_async_copy(
                rows_v.at[b], acc.at[dst_v.at[j]], ssems[b]
            ).wait()

        for p in range(nph):
            if p > 0:
                pltpu.sync_copy(srcs.at[wid, p], src_v)
                pltpu.sync_copy(dsts.at[wid, p], dst_v)
            gather(0, 0)
            gather(1, 1)

            def body(jj, carry):
                for b in range(2):
                    j = 2 * jj + b
                    step(j, b)

                    @pl.when(j + 2 < pch)
                    def _issue():
                        gather(j + 2, b)

                return carry

            lax.fori_loop(0, pch // 2, body, 0)
        plsc.subcore_barrier()

        pltpu.sync_copy(
            acc.at[pl.ds(sid * RPT, RPT)], out.at[cid, pl.ds(sid * RPT, RPT)]
        )
        if with_deg:
            pltpu.sync_copy(deg_v, out_deg.at[cid, sid])

    return agg


AGG1_NPH, AGG1_C = 4, 128
AGG2_NPH, AGG2_C = 2, 128
_agg_deg = _make_agg(True, AGG1_NPH, AGG1_C)
_agg_plain = _make_agg(False, AGG2_NPH, AGG2_C)


def _deg_to_col(degs):
    d80 = jnp.sum(degs, axis=(0, 1))
    pick = (lax.broadcasted_iota(jnp.int32, (NPAD, DR), 0) // 128
            == lax.broadcasted_iota(jnp.int32, (NPAD, DR), 1)).astype(jnp.float32)
    rows = lax.dot_general(pick, d80, (((1,), (0,)), ((), ())),
                           preferred_element_type=jnp.float32)
    lane = (lax.broadcasted_iota(jnp.int32, (NPAD, 128), 1)
            == lax.broadcasted_iota(jnp.int32, (NPAD, 128), 0) % 128)
    return jnp.sum(jnp.where(lane, rows, 0.0), axis=1, keepdims=True)


def _mm_t(a, w):
    return lax.dot_general(a, w, (((1,), (1,)), ((), ())),
                           preferred_element_type=jnp.float32)


def _root_body(a_ref, w_ref, b_ref, out_ref):
    out_ref[...] = _mm_t(a_ref[...], w_ref[...]) + b_ref[...][None, :]


def _root(a, w, b):
    return pl.pallas_call(
        _root_body,
        out_shape=jax.ShapeDtypeStruct((NPAD, 128), jnp.float32),
    )(a, w, b)


def _dense1_body(acc_ref, degs_ref, xr_ref, wl_ref, h_ref):
    s = acc_ref[0] + acc_ref[1]
    inv = 1.0 / jnp.maximum(_deg_to_col(degs_ref[...]), 1.0)
    mean = s * inv
    h = _mm_t(mean, wl_ref[...]) + xr_ref[...]
    h_ref[...] = jnp.maximum(h, 0.0)


def _dense2_body(acc_ref, degs_ref, hr_ref, wl_ref, wlin_ref, blin_ref,
                 out_ref):
    inv = 1.0 / jnp.maximum(_deg_to_col(degs_ref[...]), 1.0)
    mean = (acc_ref[0] + acc_ref[1]) * inv
    h2 = jnp.maximum(_mm_t(mean, wl_ref[...]) + hr_ref[...], 0.0)
    out_ref[...] = (_mm_t(h2, wlin_ref[...]) + blin_ref[...][None, :])[:N]


def _dense1(acc, degs, xr, Wl1):
    return pl.pallas_call(
        _dense1_body,
        out_shape=jax.ShapeDtypeStruct((NPAD, 128), jnp.float32),
    )(acc, degs, xr, Wl1)


def _dense2(acc, degs, hr, Wl2, Wlin, blin):
    return pl.pallas_call(
        _dense2_body,
        out_shape=jax.ShapeDtypeStruct((N, 128), jnp.float32),
    )(acc, degs, hr, Wl2, Wlin, blin)


def kernel(x, edge_index, Wl1, bl1, Wr1, Wl2, bl2, Wr2, Wlin, blin):
    src = edge_index[0].astype(jnp.int32)
    dst = edge_index[1].astype(jnp.int32)
    npad_e = EPP - EPW
    pad_iota = (jnp.arange(npad_e, dtype=jnp.int32)[None, :]
                + 37 * jnp.arange(NW, dtype=jnp.int32)[:, None])
    src_pad = (pad_iota * 41) % N
    dst_pad = N + pad_iota % (NPAD - N)
    srcs = jnp.concatenate([src.reshape(NW, EPW), src_pad], axis=1)
    dsts = jnp.concatenate([dst.reshape(NW, EPW), dst_pad], axis=1)

    xpad = jnp.concatenate([x, jnp.zeros((NPAD - N, 128), jnp.float32)], axis=0)
    zrows = jnp.zeros((RPT, 128), jnp.float32)

    acc1, degs = _agg_deg(
        xpad, srcs.reshape(NW, AGG1_NPH, EPP // AGG1_NPH),
        dsts.reshape(NW, AGG1_NPH, EPP // (AGG1_NPH * AGG1_C), AGG1_C), zrows)
    xr1 = _root(xpad, Wr1, bl1)
    h1 = _dense1(acc1, degs, xr1, Wl1)
    (acc2,) = _agg_plain(
        h1, srcs.reshape(NW, AGG2_NPH, EPP // AGG2_NPH),
        dsts.reshape(NW, AGG2_NPH, EPP // (AGG2_NPH * AGG2_C), AGG2_C), zrows)
    hr2 = _root(h1, Wr2, bl2)
    return _dense2(acc2, degs, hr2, Wl2, Wlin, blin)
